# baseline (speedup 1.0000x reference)
import jax
import jax.numpy as jnp
from jax import lax
from jax.experimental import pallas as pl
from jax.experimental.pallas import tpu as pltpu
from jax.experimental.pallas import tpu_sc as plsc

BATCH = 4
SEQ_LEN = 8192
D_MODEL = 768
NC, NS, L = 2, 16, 16
NW = NC * NS
ROWS_PER_W = SEQ_LEN // NW
C = 16
NCHUNK = ROWS_PER_W // C
CW = C * D_MODEL
NVEC = CW // L


def _sc_body(in_hbm, pos_hbm, out_hbm, *refs):
    inb = (refs[0:4], refs[4:8])
    posb = refs[8:10]
    sin = (refs[10:14], refs[14:18])
    sout = (refs[18:22], refs[22:26])
    spos = refs[26:28]

    wid = lax.axis_index("s") * NC + lax.axis_index("c")
    base = wid * (ROWS_PER_W * D_MODEL)

    def in_off(ci):
        return base + ci * CW

    h_in = [[None] * BATCH, [None] * BATCH]
    h_out = [[None] * BATCH, [None] * BATCH]
    h_pos = [None, None]

    h_pos[0] = pltpu.async_copy(pos_hbm.at[pl.ds(in_off(0), CW)], posb[0], spos[0])
    for b in range(BATCH):
        h_in[0][b] = pltpu.async_copy(
            in_hbm.at[b, pl.ds(in_off(0), CW)], inb[0][b], sin[0][b])

    for ci in range(NCHUNK):
        P = ci % 2
        Q = 1 - P
        if ci + 1 < NCHUNK:
            h_pos[Q] = pltpu.async_copy(
                pos_hbm.at[pl.ds(in_off(ci + 1), CW)], posb[Q], spos[Q])
            for b in range(BATCH):
                if h_out[Q][b] is not None:
                    h_out[Q][b].wait()
                h_in[Q][b] = pltpu.async_copy(
                    in_hbm.at[b, pl.ds(in_off(ci + 1), CW)], inb[Q][b], sin[Q][b])
        for b in range(BATCH):
            h_in[P][b].wait()
        h_pos[P].wait()

        dsts = inb[P]
        src = posb[P]

        @plsc.parallel_loop(0, NVEC, 1, unroll=4)
        def fused_add(v):
            sl = pl.ds(v * L, L)
            pv = src[sl]
            for b in range(BATCH):
                dsts[b][sl] = dsts[b][sl] + pv

        for b in range(BATCH):
            h_out[P][b] = pltpu.async_copy(
                dsts[b], out_hbm.at[b, pl.ds(in_off(ci), CW)], sout[P][b])

    for par in range(2):
        for b in range(BATCH):
            if h_out[par][b] is not None:
                h_out[par][b].wait()


def kernel(inputs, pos_table):
    in_flat = inputs.reshape(BATCH, SEQ_LEN * D_MODEL)
    pos_flat = pos_table.reshape(SEQ_LEN * D_MODEL)
    run = pl.kernel(
        _sc_body,
        out_type=jax.ShapeDtypeStruct((BATCH, SEQ_LEN * D_MODEL), jnp.float32),
        mesh=plsc.VectorSubcoreMesh(core_axis_name="c", subcore_axis_name="s"),
        scratch_types=(
            [pltpu.VMEM((CW,), jnp.float32) for _ in range(8)]
            + [pltpu.VMEM((CW,), jnp.float32) for _ in range(2)]
            + [pltpu.SemaphoreType.DMA for _ in range(18)]
        ),
    )
    out = run(in_flat, pos_flat)
    return out.reshape(BATCH, SEQ_LEN, D_MODEL)


# speedup vs baseline: 2.1849x; 2.1849x over previous
import jax
import jax.numpy as jnp
from jax import lax
from jax.experimental import pallas as pl
from jax.experimental.pallas import tpu as pltpu
from jax.experimental.pallas import tpu_sc as plsc

BATCH = 4
SEQ_LEN = 8192
D_MODEL = 768
NC, NS, L = 2, 16, 16
NW = NC * NS
ROWS_PER_W = SEQ_LEN // NW
C = 16
NCHUNK = ROWS_PER_W // C
VECS_PER_ROW = D_MODEL // L


def _sc_body(in_hbm, pos_hbm, out_hbm, *refs):
    inb = (refs[0:4], refs[4:8])
    posb = refs[8:10]
    sin = (refs[10:14], refs[14:18])
    sout = (refs[18:22], refs[22:26])
    spos = refs[26:28]

    wid = lax.axis_index("s") * NC + lax.axis_index("c")
    base = wid * ROWS_PER_W

    h_in = [[None] * BATCH, [None] * BATCH]
    h_out = [[None] * BATCH, [None] * BATCH]
    h_pos = [None, None]

    h_pos[0] = pltpu.async_copy(pos_hbm.at[pl.ds(base, C)], posb[0], spos[0])
    for b in range(BATCH):
        h_in[0][b] = pltpu.async_copy(
            in_hbm.at[b, pl.ds(base, C)], inb[0][b], sin[0][b])

    for ci in range(NCHUNK):
        P = ci % 2
        Q = 1 - P
        if ci + 1 < NCHUNK:
            row1 = base + (ci + 1) * C
            h_pos[Q] = pltpu.async_copy(
                pos_hbm.at[pl.ds(row1, C)], posb[Q], spos[Q])
            for b in range(BATCH):
                if h_out[Q][b] is not None:
                    h_out[Q][b].wait()
                h_in[Q][b] = pltpu.async_copy(
                    in_hbm.at[b, pl.ds(row1, C)], inb[Q][b], sin[Q][b])
        for b in range(BATCH):
            h_in[P][b].wait()
        h_pos[P].wait()

        dsts = inb[P]
        src = posb[P]

        def fused_add_row(r, _):
            for j in range(VECS_PER_ROW):
                sl = pl.ds(j * L, L)
                pv = src[r, sl]
                for b in range(BATCH):
                    dsts[b][r, sl] = dsts[b][r, sl] + pv
            return ()

        lax.fori_loop(0, C, fused_add_row, ())

        row0 = base + ci * C
        for b in range(BATCH):
            h_out[P][b] = pltpu.async_copy(
                dsts[b], out_hbm.at[b, pl.ds(row0, C)], sout[P][b])

    for par in range(2):
        for b in range(BATCH):
            if h_out[par][b] is not None:
                h_out[par][b].wait()


def kernel(inputs, pos_table):
    run = pl.kernel(
        _sc_body,
        out_type=jax.ShapeDtypeStruct((BATCH, SEQ_LEN, D_MODEL), jnp.float32),
        mesh=plsc.VectorSubcoreMesh(core_axis_name="c", subcore_axis_name="s"),
        scratch_types=(
            [pltpu.VMEM((C, D_MODEL), jnp.float32) for _ in range(8)]
            + [pltpu.VMEM((C, D_MODEL), jnp.float32) for _ in range(2)]
            + [pltpu.SemaphoreType.DMA for _ in range(18)]
        ),
    )
    return run(inputs, pos_table)


# speedup vs baseline: 2.4544x; 1.1233x over previous
import jax
import jax.numpy as jnp
from jax import lax
from jax.experimental import pallas as pl
from jax.experimental.pallas import tpu as pltpu
from jax.experimental.pallas import tpu_sc as plsc

BATCH = 4
SEQ_LEN = 8192
D_MODEL = 768
BS = 2048
TC_BATCH = 3
NC, NS, L = 2, 16, 16
NW = NC * NS
ROWS_PER_W = SEQ_LEN // NW
C = 32
NCHUNK = ROWS_PER_W // C
VECS_PER_ROW = D_MODEL // L


def _sc_body(in_hbm, pos_hbm, out_hbm,
             inb0, inb1, posb0, posb1,
             sin0, sin1, sout0, sout1, spos0, spos1):
    wid = lax.axis_index("s") * NC + lax.axis_index("c")
    base = wid * ROWS_PER_W
    inb = (inb0, inb1)
    posb = (posb0, posb1)
    sin = (sin0, sin1)
    sout = (sout0, sout1)
    spos = (spos0, spos1)

    h_in = [None, None]
    h_out = [None, None]
    h_pos = [None, None]

    h_pos[0] = pltpu.async_copy(pos_hbm.at[pl.ds(base, C)], posb[0], spos[0])
    h_in[0] = pltpu.async_copy(
        in_hbm.at[TC_BATCH, pl.ds(base, C)], inb[0], sin[0])

    for ci in range(NCHUNK):
        p = ci % 2
        q = (ci + 1) % 2
        if ci + 1 < NCHUNK:
            if h_out[q] is not None:
                h_out[q].wait()
                h_out[q] = None
            row1 = base + (ci + 1) * C
            h_in[q] = pltpu.async_copy(
                in_hbm.at[TC_BATCH, pl.ds(row1, C)], inb[q], sin[q])
            h_pos[q] = pltpu.async_copy(
                pos_hbm.at[pl.ds(row1, C)], posb[q], spos[q])
        h_in[p].wait()
        h_pos[p].wait()

        dst = inb[p]
        src = posb[p]

        def add_row(r, _):
            for j in range(VECS_PER_ROW):
                sl = pl.ds(j * L, L)
                dst[r, sl] = dst[r, sl] + src[r, sl]
            return ()

        lax.fori_loop(0, C, add_row, ())
        h_out[p] = pltpu.async_copy(
            dst, out_hbm.at[TC_BATCH, pl.ds(base + ci * C, C)], sout[p])

    h_out[0].wait()
    h_out[1].wait()


def _sc_part(inputs, pos_table):
    run = pl.kernel(
        _sc_body,
        out_type=jax.ShapeDtypeStruct((BATCH, SEQ_LEN, D_MODEL), jnp.float32),
        mesh=plsc.VectorSubcoreMesh(core_axis_name="c", subcore_axis_name="s"),
        scratch_types=[
            pltpu.VMEM((C, D_MODEL), jnp.float32),
            pltpu.VMEM((C, D_MODEL), jnp.float32),
            pltpu.VMEM((C, D_MODEL), jnp.float32),
            pltpu.VMEM((C, D_MODEL), jnp.float32),
            pltpu.SemaphoreType.DMA,
            pltpu.SemaphoreType.DMA,
            pltpu.SemaphoreType.DMA,
            pltpu.SemaphoreType.DMA,
            pltpu.SemaphoreType.DMA,
            pltpu.SemaphoreType.DMA,
        ],
    )
    return run(inputs, pos_table)


def _tc_add_kernel(x_ref, p_ref, a_ref, o_ref):
    del a_ref
    o_ref[...] = x_ref[...] + p_ref[...]


def _tc_part(inputs, pos_table, sc_out):
    grid = (SEQ_LEN // BS, TC_BATCH)
    return pl.pallas_call(
        _tc_add_kernel,
        grid=grid,
        in_specs=[
            pl.BlockSpec((1, BS, D_MODEL), lambda s, b: (b, s, 0)),
            pl.BlockSpec((BS, D_MODEL), lambda s, b: (s, 0)),
            pl.BlockSpec(memory_space=pl.ANY),
        ],
        out_specs=pl.BlockSpec((1, BS, D_MODEL), lambda s, b: (b, s, 0)),
        out_shape=jax.ShapeDtypeStruct((BATCH, SEQ_LEN, D_MODEL), jnp.float32),
        input_output_aliases={2: 0},
    )(inputs, pos_table, sc_out)


def kernel(inputs, pos_table):
    sc_out = _sc_part(inputs, pos_table)
    return _tc_part(inputs, pos_table, sc_out)


# speedup vs baseline: 2.5909x; 1.0556x over previous
import jax
import jax.numpy as jnp
from jax import lax
from jax.experimental import pallas as pl
from jax.experimental.pallas import tpu as pltpu
from jax.experimental.pallas import tpu_sc as plsc

BATCH = 4
SEQ_LEN = 8192
D_MODEL = 768
BS = 2048
SEQ_TC = 6144
NC, NS, L = 2, 16, 16
NW = NC * NS
ROWS_PER_W = (SEQ_LEN - SEQ_TC) // NW
C = 32
NCHUNK = ROWS_PER_W // C
VECS_PER_ROW = D_MODEL // L


def _sc_body(in_hbm, pos_hbm, out_hbm,
             inb0, inb1, posb0, posb1,
             sin0, sin1, sout0, sout1, spos0, spos1):
    wid = lax.axis_index("s") * NC + lax.axis_index("c")
    base = SEQ_TC + wid * ROWS_PER_W
    inb = (inb0, inb1)
    posb = (posb0, posb1)
    sin = (sin0, sin1)
    sout = (sout0, sout1)
    spos = (spos0, spos1)

    h_in = [None, None]
    h_out = [None, None]
    h_pos = [None, None]

    NSTAGE = NCHUNK * BATCH

    h_pos[0] = pltpu.async_copy(pos_hbm.at[pl.ds(base, C)], posb[0], spos[0])
    h_in[0] = pltpu.async_copy(in_hbm.at[0, pl.ds(base, C)], inb[0], sin[0])

    for k in range(NSTAGE):
        ci, b, p = k // BATCH, k % BATCH, k % 2
        q = (k + 1) % 2
        if k + 1 < NSTAGE:
            ci1, b1 = (k + 1) // BATCH, (k + 1) % BATCH
            if h_out[q] is not None:
                h_out[q].wait()
                h_out[q] = None
            h_in[q] = pltpu.async_copy(
                in_hbm.at[b1, pl.ds(base + ci1 * C, C)], inb[q], sin[q])
        if b == 0 and ci + 1 < NCHUNK:
            pp = (ci + 1) % 2
            h_pos[pp] = pltpu.async_copy(
                pos_hbm.at[pl.ds(base + (ci + 1) * C, C)], posb[pp], spos[pp])
        h_in[p].wait()
        if b == 0:
            h_pos[ci % 2].wait()

        dst = inb[p]
        src = posb[ci % 2]

        def add_row(r, _):
            for j in range(VECS_PER_ROW):
                sl = pl.ds(j * L, L)
                dst[r, sl] = dst[r, sl] + src[r, sl]
            return ()

        lax.fori_loop(0, C, add_row, ())
        h_out[p] = pltpu.async_copy(
            dst, out_hbm.at[b, pl.ds(base + ci * C, C)], sout[p])

    h_out[0].wait()
    h_out[1].wait()


def _sc_part(inputs, pos_table):
    run = pl.kernel(
        _sc_body,
        out_type=jax.ShapeDtypeStruct((BATCH, SEQ_LEN, D_MODEL), jnp.float32),
        mesh=plsc.VectorSubcoreMesh(core_axis_name="c", subcore_axis_name="s"),
        scratch_types=[
            pltpu.VMEM((C, D_MODEL), jnp.float32),
            pltpu.VMEM((C, D_MODEL), jnp.float32),
            pltpu.VMEM((C, D_MODEL), jnp.float32),
            pltpu.VMEM((C, D_MODEL), jnp.float32),
            pltpu.SemaphoreType.DMA,
            pltpu.SemaphoreType.DMA,
            pltpu.SemaphoreType.DMA,
            pltpu.SemaphoreType.DMA,
            pltpu.SemaphoreType.DMA,
            pltpu.SemaphoreType.DMA,
        ],
    )
    return run(inputs, pos_table)


def _tc_add_kernel(x_ref, p_ref, a_ref, o_ref):
    del a_ref
    o_ref[...] = x_ref[...] + p_ref[...]


def _tc_part(inputs, pos_table, sc_out):
    grid = (SEQ_TC // BS, BATCH)
    return pl.pallas_call(
        _tc_add_kernel,
        grid=grid,
        in_specs=[
            pl.BlockSpec((1, BS, D_MODEL), lambda s, b: (b, s, 0)),
            pl.BlockSpec((BS, D_MODEL), lambda s, b: (s, 0)),
            pl.BlockSpec(memory_space=pl.ANY),
        ],
        out_specs=pl.BlockSpec((1, BS, D_MODEL), lambda s, b: (b, s, 0)),
        out_shape=jax.ShapeDtypeStruct((BATCH, SEQ_LEN, D_MODEL), jnp.float32),
        input_output_aliases={2: 0},
    )(inputs, pos_table, sc_out)


def kernel(inputs, pos_table):
    sc_out = _sc_part(inputs, pos_table)
    return _tc_part(inputs, pos_table, sc_out)


# speedup vs baseline: 2.6240x; 1.0128x over previous
import jax
import jax.numpy as jnp
from jax import lax
from jax.experimental import pallas as pl
from jax.experimental.pallas import tpu as pltpu
from jax.experimental.pallas import tpu_sc as plsc

BATCH = 4
SEQ_LEN = 8192
D_MODEL = 768
BS = 1024
SEQ_TC = 7168
NC, NS, L = 2, 16, 16
NW = NC * NS
ROWS_PER_W = (SEQ_LEN - SEQ_TC) // NW
C = 32
NCHUNK = ROWS_PER_W // C
VECS_PER_ROW = D_MODEL // L


def _sc_body(in_hbm, pos_hbm, out_hbm,
             inb0, inb1, posb0, posb1,
             sin0, sin1, sout0, sout1, spos0, spos1):
    wid = lax.axis_index("s") * NC + lax.axis_index("c")
    base = SEQ_TC + wid * ROWS_PER_W
    inb = (inb0, inb1)
    posb = (posb0, posb1)
    sin = (sin0, sin1)
    sout = (sout0, sout1)
    spos = (spos0, spos1)

    h_in = [None, None]
    h_out = [None, None]
    h_pos = [None, None]

    NSTAGE = NCHUNK * BATCH

    h_pos[0] = pltpu.async_copy(pos_hbm.at[pl.ds(base, C)], posb[0], spos[0])
    h_in[0] = pltpu.async_copy(in_hbm.at[0, pl.ds(base, C)], inb[0], sin[0])

    for k in range(NSTAGE):
        ci, b, p = k // BATCH, k % BATCH, k % 2
        q = (k + 1) % 2
        if k + 1 < NSTAGE:
            ci1, b1 = (k + 1) // BATCH, (k + 1) % BATCH
            if h_out[q] is not None:
                h_out[q].wait()
                h_out[q] = None
            h_in[q] = pltpu.async_copy(
                in_hbm.at[b1, pl.ds(base + ci1 * C, C)], inb[q], sin[q])
        if b == 0 and ci + 1 < NCHUNK:
            pp = (ci + 1) % 2
            h_pos[pp] = pltpu.async_copy(
                pos_hbm.at[pl.ds(base + (ci + 1) * C, C)], posb[pp], spos[pp])
        h_in[p].wait()
        if b == 0:
            h_pos[ci % 2].wait()

        dst = inb[p]
        src = posb[ci % 2]

        def add_row(r, _):
            for j in range(VECS_PER_ROW):
                sl = pl.ds(j * L, L)
                dst[r, sl] = dst[r, sl] + src[r, sl]
            return ()

        lax.fori_loop(0, C, add_row, ())
        h_out[p] = pltpu.async_copy(
            dst, out_hbm.at[b, pl.ds(base + ci * C, C)], sout[p])

    h_out[0].wait()
    h_out[1].wait()


def _sc_part(inputs, pos_table):
    run = pl.kernel(
        _sc_body,
        out_type=jax.ShapeDtypeStruct((BATCH, SEQ_LEN, D_MODEL), jnp.float32),
        mesh=plsc.VectorSubcoreMesh(core_axis_name="c", subcore_axis_name="s"),
        scratch_types=[
            pltpu.VMEM((C, D_MODEL), jnp.float32),
            pltpu.VMEM((C, D_MODEL), jnp.float32),
            pltpu.VMEM((C, D_MODEL), jnp.float32),
            pltpu.VMEM((C, D_MODEL), jnp.float32),
            pltpu.SemaphoreType.DMA,
            pltpu.SemaphoreType.DMA,
            pltpu.SemaphoreType.DMA,
            pltpu.SemaphoreType.DMA,
            pltpu.SemaphoreType.DMA,
            pltpu.SemaphoreType.DMA,
        ],
    )
    return run(inputs, pos_table)


def _tc_add_kernel(x_ref, p_ref, a_ref, o_ref):
    del a_ref
    o_ref[...] = x_ref[...] + p_ref[...]


def _tc_part(inputs, pos_table, sc_out):
    grid = (SEQ_TC // BS, BATCH)
    return pl.pallas_call(
        _tc_add_kernel,
        grid=grid,
        in_specs=[
            pl.BlockSpec((1, BS, D_MODEL), lambda s, b: (b, s, 0)),
            pl.BlockSpec((BS, D_MODEL), lambda s, b: (s, 0)),
            pl.BlockSpec(memory_space=pl.ANY),
        ],
        out_specs=pl.BlockSpec((1, BS, D_MODEL), lambda s, b: (b, s, 0)),
        out_shape=jax.ShapeDtypeStruct((BATCH, SEQ_LEN, D_MODEL), jnp.float32),
        input_output_aliases={2: 0},
    )(inputs, pos_table, sc_out)


def kernel(inputs, pos_table):
    sc_out = _sc_part(inputs, pos_table)
    return _tc_part(inputs, pos_table, sc_out)
